# Initial kernel scaffold; baseline (speedup 1.0000x reference)
import jax
import jax.numpy as jnp
from jax.experimental import pallas as pl


def kernel(pc1, pc2, feature1, feature2, W0, b0, W1, b1, W2, b2, Wa1, ba1, g1, be1, Wb1, bb1, Wa2, ba2, g2, be2, Wb2, bb2):
    raise NotImplementedError("write your pallas kernel here")


# speedup vs baseline: 11.5034x; 11.5034x over previous
import functools

import jax
import jax.numpy as jnp
from jax import lax
from jax.experimental import pallas as pl
from jax.experimental.pallas import tpu as pltpu
from jax.experimental.pallas import tpu_sc as plsc

N = 4096
K_NN = 16
C = 256
XP = 16
XG = 128
BLK = 256
NBLK = N // BLK
R = BLK * K_NN
F32 = jnp.float32

SC_NC = 2
SC_NS = 16
SC_NW = SC_NC * SC_NS
GCH = 256


def _leaky(x):
    return jnp.where(x > 0, x, 0.1 * x)


def _prep_body(f1_ref, f2_ref, w0at_ref, w0bt_ref, a1_ref, a2_ref):
    a1_ref[...] = jnp.dot(f1_ref[...], w0at_ref[...], preferred_element_type=F32)
    a2_ref[...] = jnp.dot(f2_ref[...], w0bt_ref[...], preferred_element_type=F32)


def _prep(f1_t, f2_t, w0at, w0bt):
    return pl.pallas_call(
        _prep_body,
        out_shape=[jax.ShapeDtypeStruct((N, C), F32),
                   jax.ShapeDtypeStruct((N, C), F32)],
    )(f1_t, f2_t, w0at, w0bt)


def _knn_body(q_ref, kt_ref, idx_ref):
    q = q_ref[...]
    kt = kt_ref[...]
    qk = jnp.dot(q, kt, preferred_element_type=F32)
    q2 = jnp.sum(q * q, axis=1, keepdims=True)
    k2 = jnp.sum(kt * kt, axis=0, keepdims=True)
    d2 = q2 + k2 - 2.0 * qk
    iota = lax.broadcasted_iota(jnp.int32, d2.shape, 1)
    big = jnp.int32(2 ** 30)
    for k in range(K_NN):
        minv = jnp.min(d2, axis=1, keepdims=True)
        sel = jnp.where(d2 == minv, iota, big)
        idxk = jnp.min(sel, axis=1)
        idx_ref[:, k] = idxk
        d2 = jnp.where(iota == idxk[:, None], jnp.float32(jnp.inf), d2)


def _knn(q8, kt8):
    return pl.pallas_call(
        _knn_body,
        grid=(NBLK,),
        in_specs=[pl.BlockSpec((BLK, 8), lambda i: (i, 0)),
                  pl.BlockSpec((8, N), lambda i: (0, 0))],
        out_specs=pl.BlockSpec((BLK, K_NN), lambda i: (i, 0)),
        out_shape=jax.ShapeDtypeStruct((N, K_NN), jnp.int32),
    )(q8, kt8)


def _sc_gather_body(a2_hbm, xyz_hbm, idx_hbm, gfeat_hbm, gxyz_hbm,
                    idx_v, rows_v, rows2_v, sem, sem2):
    wid = lax.axis_index("s") * SC_NC + lax.axis_index("c")
    b_per_w = (N * K_NN) // SC_NW
    base = wid * b_per_w
    for ch in range(b_per_w // GCH):
        off = base + ch * GCH
        pltpu.sync_copy(idx_hbm.at[pl.ds(off, GCH)], idx_v)
        cp1 = pltpu.async_copy(a2_hbm.at[idx_v], rows_v, sem)
        cp2 = pltpu.async_copy(xyz_hbm.at[idx_v], rows2_v, sem2)
        cp1.wait()
        cp2.wait()
        pltpu.sync_copy(rows_v, gfeat_hbm.at[pl.ds(off, GCH)])
        pltpu.sync_copy(rows2_v, gxyz_hbm.at[pl.ds(off, GCH)])


def _sc_gather(a2, xyz_t, idx_flat):
    mesh = plsc.VectorSubcoreMesh(core_axis_name="c", subcore_axis_name="s")
    fn = pl.kernel(
        _sc_gather_body,
        out_type=[jax.ShapeDtypeStruct((N * K_NN, C), F32),
                  jax.ShapeDtypeStruct((N * K_NN, XG), F32)],
        mesh=mesh,
        scratch_types=[pltpu.VMEM((GCH,), jnp.int32),
                       pltpu.VMEM((GCH, C), F32),
                       pltpu.VMEM((GCH, XG), F32),
                       pltpu.SemaphoreType.DMA,
                       pltpu.SemaphoreType.DMA],
    )
    return fn(a2, xyz_t, idx_flat)


def _stats_body(gxyz_ref, qrep_ref, dir_ref, s1_ref, s2_ref):
    i = pl.program_id(0)
    d = gxyz_ref[:, :XP] - qrep_ref[...]
    dir_ref[...] = d
    s1 = jnp.sum(d, axis=0, keepdims=True)
    s2 = lax.dot_general(d, d, (((0,), (0,)), ((), ())),
                         preferred_element_type=F32)

    @pl.when(i == 0)
    def _():
        s1_ref[...] = s1
        s2_ref[...] = s2

    @pl.when(i > 0)
    def _():
        s1_ref[...] += s1
        s2_ref[...] += s2


def _stats(gxyz, qrep):
    return pl.pallas_call(
        _stats_body,
        grid=(NBLK,),
        in_specs=[pl.BlockSpec((R, XG), lambda i: (i, 0)),
                  pl.BlockSpec((R, XP), lambda i: (i, 0))],
        out_specs=[pl.BlockSpec((R, XP), lambda i: (i, 0)),
                   pl.BlockSpec((1, XP), lambda i: (0, 0)),
                   pl.BlockSpec((XP, XP), lambda i: (0, 0))],
        out_shape=[jax.ShapeDtypeStruct((N * K_NN, XP), F32),
                   jax.ShapeDtypeStruct((1, XP), F32),
                   jax.ShapeDtypeStruct((XP, XP), F32)],
    )(gxyz, qrep)


def _weightnet_rows(dirv, s1, s2, wap, grow, berow, wbt, bbrow):
    nk = jnp.float32(N * K_NN)
    m = s1 / nk
    mm = lax.dot_general(m, m, (((0,), (0,)), ((), ())),
                         preferred_element_type=F32)
    cov = s2 / nk - mm
    murow = lax.dot_general(m, wap, (((1,), (1,)), ((), ())),
                            preferred_element_type=F32)
    wac = jnp.dot(wap, cov, preferred_element_type=F32)
    ones = jnp.ones((1, XP), F32)
    varrow = lax.dot_general(ones, wap * wac, (((1,), (1,)), ((), ())),
                             preferred_element_type=F32)
    scalerow = grow * lax.rsqrt(varrow + 1e-5)
    beff = berow - murow * scalerow
    hw = lax.dot_general(dirv, wap, (((1,), (1,)), ((), ())),
                         preferred_element_type=F32)
    hw = jnp.maximum(hw * scalerow + beff, 0.0)
    return jnp.dot(hw, wbt, preferred_element_type=F32) + bbrow


def _ksum(p):
    p3 = p.reshape(BLK, K_NN, C)
    acc = p3[:, 0, :]
    for k in range(1, K_NN):
        acc = acc + p3[:, k, :]
    return acc


def _mlp_body(a1_ref, g_ref, dir_ref, s1_ref, s2_ref, w0dt_ref, b0_ref,
              w1t_ref, b1_ref, w2t_ref, b2_ref,
              wap_ref, g1_ref, be1_ref, wb1t_ref, bb1_ref, out_ref):
    dirv = dir_ref[...]
    a1 = a1_ref[...]
    a1b = jnp.broadcast_to(a1[:, None, :], (BLK, K_NN, C)).reshape(R, C)
    h0 = g_ref[...] + a1b + b0_ref[...]
    h0 = h0 + jnp.dot(dirv, w0dt_ref[...], preferred_element_type=F32)
    x = _leaky(h0)
    x = _leaky(jnp.dot(x, w1t_ref[...], preferred_element_type=F32) + b1_ref[...])
    x = _leaky(jnp.dot(x, w2t_ref[...], preferred_element_type=F32) + b2_ref[...])
    w = _weightnet_rows(dirv, s1_ref[...], s2_ref[...], wap_ref[...],
                        g1_ref[...], be1_ref[...], wb1t_ref[...], bb1_ref[...])
    out_ref[...] = _ksum(w * x)


def _mlp(a1, g, dirv, s1, s2, w0dt, b0r, w1t, b1r, w2t, b2r,
         wap, g1r, be1r, wb1t, bb1r):
    full = lambda shape: pl.BlockSpec(shape, lambda i: tuple(0 for _ in shape))
    return pl.pallas_call(
        _mlp_body,
        grid=(NBLK,),
        in_specs=[pl.BlockSpec((BLK, C), lambda i: (i, 0)),
                  pl.BlockSpec((R, C), lambda i: (i, 0)),
                  pl.BlockSpec((R, XP), lambda i: (i, 0)),
                  full((1, XP)), full((XP, XP)), full((XP, C)), full((1, C)),
                  full((C, C)), full((1, C)), full((C, C)), full((1, C)),
                  full((8, XP)), full((1, 8)), full((1, 8)),
                  full((8, C)), full((1, C))],
        out_specs=pl.BlockSpec((BLK, C), lambda i: (i, 0)),
        out_shape=jax.ShapeDtypeStruct((N, C), F32),
    )(a1, g, dirv, s1, s2, w0dt, b0r, w1t, b1r, w2t, b2r,
      wap, g1r, be1r, wb1t, bb1r)


def _agg_body(gx_ref, dir_ref, s1_ref, s2_ref,
              wap_ref, g2_ref, be2_ref, wb2t_ref, bb2_ref, out_ref):
    dirv = dir_ref[...]
    w = _weightnet_rows(dirv, s1_ref[...], s2_ref[...], wap_ref[...],
                        g2_ref[...], be2_ref[...], wb2t_ref[...], bb2_ref[...])
    out_ref[...] = _ksum(w * gx_ref[...])


def _agg(gx, dirv, s1, s2, wap, g2r, be2r, wb2t, bb2r):
    full = lambda shape: pl.BlockSpec(shape, lambda i: tuple(0 for _ in shape))
    return pl.pallas_call(
        _agg_body,
        grid=(NBLK,),
        in_specs=[pl.BlockSpec((R, C), lambda i: (i, 0)),
                  pl.BlockSpec((R, XP), lambda i: (i, 0)),
                  full((1, XP)), full((XP, XP)),
                  full((8, XP)), full((1, 8)), full((1, 8)),
                  full((8, C)), full((1, C))],
        out_specs=pl.BlockSpec((BLK, C), lambda i: (i, 0)),
        out_shape=jax.ShapeDtypeStruct((N, C), F32),
    )(gx, dirv, s1, s2, wap, g2r, be2r, wb2t, bb2r)


def kernel(pc1, pc2, feature1, feature2, W0, b0, W1, b1, W2, b2,
           Wa1, ba1, g1, be1, Wb1, bb1, Wa2, ba2, g2, be2, Wb2, bb2):
    q1 = jnp.transpose(pc1[0])
    q1p8 = jnp.concatenate([q1, jnp.zeros((N, 5), F32)], axis=1)
    q1p16 = jnp.concatenate([q1, jnp.zeros((N, XP - 3), F32)], axis=1)
    q1p128 = jnp.concatenate([q1, jnp.zeros((N, XG - 3), F32)], axis=1)
    kt1p8 = jnp.concatenate([pc1[0], jnp.zeros((5, N), F32)], axis=0)
    kt2p8 = jnp.concatenate([pc2[0], jnp.zeros((5, N), F32)], axis=0)
    q2 = jnp.transpose(pc2[0])
    q2p128 = jnp.concatenate([q2, jnp.zeros((N, XG - 3), F32)], axis=1)
    f1_t = jnp.transpose(feature1[0])
    f2_t = jnp.transpose(feature2[0])
    w0at = jnp.transpose(W0[:, :C])
    w0bt = jnp.transpose(W0[:, C:2 * C])
    w0dt = jnp.concatenate([jnp.transpose(W0[:, 2 * C:]),
                            jnp.zeros((XP - 3, C), F32)], axis=0)
    wa1p = jnp.concatenate([Wa1, jnp.zeros((8, XP - 3), F32)], axis=1)
    wa2p = jnp.concatenate([Wa2, jnp.zeros((8, XP - 3), F32)], axis=1)
    qrep1 = jnp.repeat(q1p16, K_NN, axis=0)
    row = lambda v: v.reshape(1, -1)

    a1, a2 = _prep(f1_t, f2_t, w0at, w0bt)
    idx1 = _knn(q1p8, kt2p8)
    gfeat, gxyz = _sc_gather(a2, q2p128, idx1.reshape(-1))
    dir1, s1a, s2a = _stats(gxyz, qrep1)
    x = _mlp(a1, gfeat, dir1, s1a, s2a, w0dt, row(b0), jnp.transpose(W1),
             row(b1), jnp.transpose(W2), row(b2),
             wa1p, row(g1), row(be1), jnp.transpose(Wb1), row(bb1))

    idx2 = _knn(q1p8, kt1p8)
    gx, gxyz2 = _sc_gather(x, q1p128, idx2.reshape(-1))
    dir2, s1b, s2b = _stats(gxyz2, qrep1)
    out = _agg(gx, dir2, s1b, s2b, wa2p, row(g2), row(be2),
               jnp.transpose(Wb2), row(bb2))
    return jnp.transpose(out)[None, :, :]
